# Initial kernel scaffold; baseline (speedup 1.0000x reference)
import jax
import jax.numpy as jnp
from jax.experimental import pallas as pl


def kernel(locs, init_embeddings, W, b):
    raise NotImplementedError("write your pallas kernel here")


# speedup vs baseline: 3.4980x; 3.4980x over previous
import jax
import jax.numpy as jnp
from jax import lax
from jax.experimental import pallas as pl
from jax.experimental.pallas import tpu as pltpu

EMBED = 128
K = 32
ROWS = 256


def _edge_kernel(locs_ref, locsT_ref, sel_ref, wt_ref, bt_ref,
                 src_ref, dst_ref, emb_ref, cur_ref):
    b = pl.program_id(0)
    t = pl.program_id(1)
    R = src_ref.shape[0]
    N = locsT_ref.shape[2]

    pts = locs_ref[0]
    ptsT = locsT_ref[0]
    xr = pts[:, 0:1]
    yr = pts[:, 1:2]
    xa = ptsT[0:1, :]
    ya = ptsT[1:2, :]
    sq_r = xr * xr + yr * yr
    sq_a = xa * xa + ya * ya
    dot = lax.dot_general(pts, ptsT, (((1,), (0,)), ((), ())),
                          preferred_element_type=jnp.float32)
    d2 = (sq_r + sq_a) - 2.0 * dot
    dist = jnp.sqrt(jnp.maximum(d2, 1e-12))

    col = lax.broadcasted_iota(jnp.int32, (R, N), 1)
    rowg = lax.broadcasted_iota(jnp.int32, (R, N), 0) + t * R
    cur_ref[...] = jnp.where(col == rowg, jnp.inf, dist)

    lane_k = lax.broadcasted_iota(jnp.int32, (R, K), 1)

    def body(j, carry):
        vals_acc, idx_acc = carry
        a = cur_ref[...]
        m = jnp.min(a, axis=1, keepdims=True)
        cand = jnp.where(a == m, col, N)
        idx = jnp.min(cand, axis=1, keepdims=True)
        cur_ref[...] = jnp.where(col == idx, jnp.inf, a)
        vals_acc = jnp.where(lane_k == j, m, vals_acc)
        idx_acc = jnp.where(lane_k == j, idx, idx_acc)
        return vals_acc, idx_acc

    vals, idxs = lax.fori_loop(
        0, K, body,
        (jnp.zeros((R, K), jnp.float32), jnp.zeros((R, K), jnp.int32)))

    row_ids = lax.broadcasted_iota(jnp.int32, (R, K), 0) + (b * N + t * R)
    src_ref[...] = row_ids
    dst_ref[...] = idxs + b * N
    emb = lax.dot_general(vals, sel_ref[...], (((1,), (0,)), ((), ())),
                          preferred_element_type=jnp.float32)
    emb_ref[...] = emb * wt_ref[...] + bt_ref[...]


def kernel(locs, init_embeddings, W, b):
    Bb, Nn, _ = locs.shape
    NT = Nn // ROWS
    locsT = jnp.transpose(locs, (0, 2, 1))
    wt = jnp.tile(W[:, 0], K).reshape(1, K * EMBED)
    bt = jnp.tile(b, K).reshape(1, K * EMBED)
    sel = (jnp.arange(K * EMBED, dtype=jnp.int32) // EMBED
           == jnp.arange(K, dtype=jnp.int32)[:, None]).astype(jnp.float32)

    src, dst, emb = pl.pallas_call(
        _edge_kernel,
        grid=(Bb, NT),
        in_specs=[
            pl.BlockSpec((1, ROWS, 2), lambda b_, t: (b_, t, 0)),
            pl.BlockSpec((1, 2, Nn), lambda b_, t: (b_, 0, 0)),
            pl.BlockSpec((K, K * EMBED), lambda b_, t: (0, 0)),
            pl.BlockSpec((1, K * EMBED), lambda b_, t: (0, 0)),
            pl.BlockSpec((1, K * EMBED), lambda b_, t: (0, 0)),
        ],
        out_specs=[
            pl.BlockSpec((ROWS, K), lambda b_, t: (b_ * (Nn // ROWS) + t, 0)),
            pl.BlockSpec((ROWS, K), lambda b_, t: (b_ * (Nn // ROWS) + t, 0)),
            pl.BlockSpec((ROWS, K * EMBED),
                         lambda b_, t: (b_ * (Nn // ROWS) + t, 0)),
        ],
        out_shape=[
            jax.ShapeDtypeStruct((Bb * Nn, K), jnp.int32),
            jax.ShapeDtypeStruct((Bb * Nn, K), jnp.int32),
            jax.ShapeDtypeStruct((Bb * Nn, K * EMBED), jnp.float32),
        ],
        scratch_shapes=[pltpu.VMEM((ROWS, Nn), jnp.float32)],
    )(locs, locsT, sel, wt, bt)

    edge_index = jnp.stack([src.reshape(-1), dst.reshape(-1)], axis=0)
    edge_emb = emb.reshape(Bb * Nn * K, EMBED)
    x = init_embeddings.reshape(Bb * Nn, EMBED)
    return x, edge_index, edge_emb


# speedup vs baseline: 4.6394x; 1.3263x over previous
import jax
import jax.numpy as jnp
from jax import lax
from jax.experimental import pallas as pl
from jax.experimental.pallas import tpu as pltpu

EMBED = 128
K = 32
ROWS = 256


def _edge_kernel(locs_ref, locsT_ref, sel_ref, wt_ref, bt_ref,
                 src_ref, dst_ref, emb_ref, cur_ref):
    b = pl.program_id(0)
    t = pl.program_id(1)
    R = src_ref.shape[0]
    N = locsT_ref.shape[2]

    pts = locs_ref[0]
    ptsT = locsT_ref[0]
    xr = pts[:, 0:1]
    yr = pts[:, 1:2]
    xa = ptsT[0:1, :]
    ya = ptsT[1:2, :]
    sq_r = xr * xr + yr * yr
    sq_a = xa * xa + ya * ya
    dot = lax.dot_general(pts, ptsT, (((1,), (0,)), ((), ())),
                          preferred_element_type=jnp.float32)
    d2 = (sq_r + sq_a) - 2.0 * dot
    dist = jnp.sqrt(jnp.maximum(d2, 1e-12))

    col = lax.broadcasted_iota(jnp.int32, (R, N), 1)
    colf = lax.broadcasted_iota(jnp.int32, (R, N), 1).astype(jnp.float32)
    rowg = lax.broadcasted_iota(jnp.int32, (R, N), 0) + t * R
    cur_ref[...] = jnp.where(col == rowg, jnp.inf, dist)

    lane_k = lax.broadcasted_iota(jnp.int32, (R, K), 1)
    big = jnp.float32(N)

    def body(j, carry):
        vals_acc, idx_acc = carry
        a = cur_ref[...]
        m = jnp.min(a, axis=1, keepdims=True)
        cand = jnp.where(a == m, colf, big)
        idx = jnp.min(cand, axis=1, keepdims=True)
        cur_ref[...] = jnp.where(colf == idx, jnp.inf, a)
        vals_acc = jnp.where(lane_k == j, m, vals_acc)
        idx_acc = jnp.where(lane_k == j, idx, idx_acc)
        return vals_acc, idx_acc

    vals, idxf = lax.fori_loop(
        0, K - 1, body,
        (jnp.zeros((R, K), jnp.float32), jnp.zeros((R, K), jnp.float32)))

    a = cur_ref[...]
    m = jnp.min(a, axis=1, keepdims=True)
    cand = jnp.where(a == m, colf, big)
    idx = jnp.min(cand, axis=1, keepdims=True)
    vals = jnp.where(lane_k == (K - 1), m, vals)
    idxf = jnp.where(lane_k == (K - 1), idx, idxf)

    row_ids = lax.broadcasted_iota(jnp.int32, (R, K), 0) + (b * N + t * R)
    src_ref[...] = row_ids
    dst_ref[...] = idxf.astype(jnp.int32) + b * N
    emb = lax.dot_general(vals, sel_ref[...], (((1,), (0,)), ((), ())),
                          preferred_element_type=jnp.float32)
    emb_ref[...] = emb * wt_ref[...] + bt_ref[...]


def kernel(locs, init_embeddings, W, b):
    Bb, Nn, _ = locs.shape
    NT = Nn // ROWS
    locsT = jnp.transpose(locs, (0, 2, 1))
    wt = jnp.tile(W[:, 0], K).reshape(1, K * EMBED)
    bt = jnp.tile(b, K).reshape(1, K * EMBED)
    sel = (jnp.arange(K * EMBED, dtype=jnp.int32) // EMBED
           == jnp.arange(K, dtype=jnp.int32)[:, None]).astype(jnp.float32)

    src, dst, emb = pl.pallas_call(
        _edge_kernel,
        grid=(Bb, NT),
        in_specs=[
            pl.BlockSpec((1, ROWS, 2), lambda b_, t: (b_, t, 0)),
            pl.BlockSpec((1, 2, Nn), lambda b_, t: (b_, 0, 0)),
            pl.BlockSpec((K, K * EMBED), lambda b_, t: (0, 0)),
            pl.BlockSpec((1, K * EMBED), lambda b_, t: (0, 0)),
            pl.BlockSpec((1, K * EMBED), lambda b_, t: (0, 0)),
        ],
        out_specs=[
            pl.BlockSpec((ROWS, K), lambda b_, t: (b_ * (Nn // ROWS) + t, 0)),
            pl.BlockSpec((ROWS, K), lambda b_, t: (b_ * (Nn // ROWS) + t, 0)),
            pl.BlockSpec((ROWS, K * EMBED),
                         lambda b_, t: (b_ * (Nn // ROWS) + t, 0)),
        ],
        out_shape=[
            jax.ShapeDtypeStruct((Bb * Nn, K), jnp.int32),
            jax.ShapeDtypeStruct((Bb * Nn, K), jnp.int32),
            jax.ShapeDtypeStruct((Bb * Nn, K * EMBED), jnp.float32),
        ],
        scratch_shapes=[pltpu.VMEM((ROWS, Nn), jnp.float32)],
    )(locs, locsT, sel, wt, bt)

    edge_index = jnp.stack([src.reshape(-1), dst.reshape(-1)], axis=0)
    edge_emb = emb.reshape(Bb * Nn * K, EMBED)
    x = init_embeddings.reshape(Bb * Nn, EMBED)
    return x, edge_index, edge_emb


# speedup vs baseline: 4.6591x; 1.0042x over previous
import jax
import jax.numpy as jnp
from jax import lax
from jax.experimental import pallas as pl
from jax.experimental.pallas import tpu as pltpu

EMBED = 128
K = 32
ROWS = 512


def _edge_kernel(locs_ref, locsT_ref, sel_ref, wt_ref, bt_ref,
                 src_ref, dst_ref, emb_ref, cur_ref):
    b = pl.program_id(0)
    t = pl.program_id(1)
    R = src_ref.shape[0]
    N = locsT_ref.shape[2]

    pts = locs_ref[0]
    ptsT = locsT_ref[0]
    xr = pts[:, 0:1]
    yr = pts[:, 1:2]
    xa = ptsT[0:1, :]
    ya = ptsT[1:2, :]
    sq_r = xr * xr + yr * yr
    sq_a = xa * xa + ya * ya
    dot = lax.dot_general(pts, ptsT, (((1,), (0,)), ((), ())),
                          preferred_element_type=jnp.float32)
    d2 = (sq_r + sq_a) - 2.0 * dot
    dist = jnp.sqrt(jnp.maximum(d2, 1e-12))

    col = lax.broadcasted_iota(jnp.int32, (R, N), 1)
    colf = lax.broadcasted_iota(jnp.int32, (R, N), 1).astype(jnp.float32)
    rowg = lax.broadcasted_iota(jnp.int32, (R, N), 0) + t * R
    cur_ref[...] = jnp.where(col == rowg, jnp.inf, dist)

    lane_k = lax.broadcasted_iota(jnp.int32, (R, K), 1)
    big = jnp.float32(N)

    def body(j, carry):
        vals_acc, idx_acc = carry
        a = cur_ref[...]
        m = jnp.min(a, axis=1, keepdims=True)
        cand = jnp.where(a == m, colf, big)
        idx = jnp.min(cand, axis=1, keepdims=True)
        cur_ref[...] = jnp.where(colf == idx, jnp.inf, a)
        vals_acc = jnp.where(lane_k == j, m, vals_acc)
        idx_acc = jnp.where(lane_k == j, idx, idx_acc)
        return vals_acc, idx_acc

    vals, idxf = lax.fori_loop(
        0, K - 1, body,
        (jnp.zeros((R, K), jnp.float32), jnp.zeros((R, K), jnp.float32)))

    a = cur_ref[...]
    m = jnp.min(a, axis=1, keepdims=True)
    cand = jnp.where(a == m, colf, big)
    idx = jnp.min(cand, axis=1, keepdims=True)
    vals = jnp.where(lane_k == (K - 1), m, vals)
    idxf = jnp.where(lane_k == (K - 1), idx, idxf)

    row_ids = lax.broadcasted_iota(jnp.int32, (R, K), 0) + (b * N + t * R)
    src_ref[...] = row_ids
    dst_ref[...] = idxf.astype(jnp.int32) + b * N
    emb = lax.dot_general(vals, sel_ref[...], (((1,), (0,)), ((), ())),
                          preferred_element_type=jnp.float32)
    emb_ref[...] = emb * wt_ref[...] + bt_ref[...]


def kernel(locs, init_embeddings, W, b):
    Bb, Nn, _ = locs.shape
    NT = Nn // ROWS
    locsT = jnp.transpose(locs, (0, 2, 1))
    wt = jnp.tile(W[:, 0], K).reshape(1, K * EMBED)
    bt = jnp.tile(b, K).reshape(1, K * EMBED)
    sel = (jnp.arange(K * EMBED, dtype=jnp.int32) // EMBED
           == jnp.arange(K, dtype=jnp.int32)[:, None]).astype(jnp.float32)

    src, dst, emb = pl.pallas_call(
        _edge_kernel,
        grid=(Bb, NT),
        in_specs=[
            pl.BlockSpec((1, ROWS, 2), lambda b_, t: (b_, t, 0)),
            pl.BlockSpec((1, 2, Nn), lambda b_, t: (b_, 0, 0)),
            pl.BlockSpec((K, K * EMBED), lambda b_, t: (0, 0)),
            pl.BlockSpec((1, K * EMBED), lambda b_, t: (0, 0)),
            pl.BlockSpec((1, K * EMBED), lambda b_, t: (0, 0)),
        ],
        out_specs=[
            pl.BlockSpec((ROWS, K), lambda b_, t: (b_ * (Nn // ROWS) + t, 0)),
            pl.BlockSpec((ROWS, K), lambda b_, t: (b_ * (Nn // ROWS) + t, 0)),
            pl.BlockSpec((ROWS, K * EMBED),
                         lambda b_, t: (b_ * (Nn // ROWS) + t, 0)),
        ],
        out_shape=[
            jax.ShapeDtypeStruct((Bb * Nn, K), jnp.int32),
            jax.ShapeDtypeStruct((Bb * Nn, K), jnp.int32),
            jax.ShapeDtypeStruct((Bb * Nn, K * EMBED), jnp.float32),
        ],
        scratch_shapes=[pltpu.VMEM((ROWS, Nn), jnp.float32)],
    )(locs, locsT, sel, wt, bt)

    edge_index = jnp.stack([src.reshape(-1), dst.reshape(-1)], axis=0)
    edge_emb = emb.reshape(Bb * Nn * K, EMBED)
    x = init_embeddings.reshape(Bb * Nn, EMBED)
    return x, edge_index, edge_emb
